# Initial kernel scaffold; baseline (speedup 1.0000x reference)
import jax
import jax.numpy as jnp
from jax.experimental import pallas as pl


def kernel(user_table, item_table, edge_index):
    raise NotImplementedError("write your pallas kernel here")


# speedup vs baseline: 142.4223x; 142.4223x over previous
import functools

import jax
import jax.numpy as jnp
from jax import lax
from jax.experimental import pallas as pl
from jax.experimental.pallas import tpu as pltpu
from jax.experimental.pallas import tpu_sc as plsc

NU = 50000
NI = 50000
EMB = 16
NC = 2
NS = 16
L = 16

RPT = 3200
NPAD = NS * RPT
RCHUNK = 320
NCHUNKS = RPT // RCHUNK

_f32 = jnp.float32
_i32 = jnp.int32

_mesh = plsc.VectorSubcoreMesh(
    core_axis_name="c", subcore_axis_name="s", num_cores=NC, num_subcores=NS
)


def _zero_vec(ref, nwords):
    z16 = jnp.zeros((L,), _f32)

    def zz(i, _):
        ref[pl.ds(i * L, L)] = z16
        return 0

    lax.fori_loop(0, nwords // L, zz, 0)


def _fast_rsqrt(x):
    xi = plsc.bitcast(x, _i32)
    xi = jnp.int32(0x5F3759DF) - (xi >> 1)
    y = plsc.bitcast(xi, _f32)
    hx = x * jnp.float32(0.5)
    y = y * (jnp.float32(1.5) - hx * y * y)
    y = y * (jnp.float32(1.5) - hx * y * y)
    y = y * (jnp.float32(1.5) - hx * y * y)
    return y


def _deg_z0_body(ut, it, u2, v2, z0u, z0i, dsqu, dsqi, squ, sqi,
                 deg_parts, hist, idxb, rowb, stage, dslice, pbuf, sqbuf,
                 ybuf, tbuf):
    c = lax.axis_index("c")
    s = lax.axis_index("s")
    one16 = jnp.ones((L,), _f32)

    _zero_vec(hist, NPAD)

    def histogram(dst2):
        ept = (dst2.shape[0] - 1024) // NS
        base = s * ept

        def chunk(g, _):
            pltpu.sync_copy(dst2.at[pl.ds(base + g * 1024, 1024)], idxb)

            def kloop(kk, _):
                idx = idxb[pl.ds(kk * L, L)]
                plsc.addupdate_scatter(hist, [idx], one16)
                return 0

            lax.fori_loop(0, 1024 // L, kloop, 0)
            return 0

        lax.fori_loop(0, ept // 1024, chunk, 0)

    @pl.when(c == 0)
    def _():
        histogram(u2)

    @pl.when(c == 1)
    def _():
        histogram(v2)

    pltpu.sync_copy(hist, deg_parts.at[s])
    plsc.subcore_barrier()

    pltpu.sync_copy(deg_parts.at[0, pl.ds(s * RPT, RPT)], dslice)

    def accum(t, _):
        pltpu.sync_copy(deg_parts.at[t, pl.ds(s * RPT, RPT)], pbuf)

        def avec(i, _):
            dslice[pl.ds(i * L, L)] = (
                dslice[pl.ds(i * L, L)] + pbuf[pl.ds(i * L, L)]
            )
            return 0

        lax.fori_loop(0, RPT // L, avec, 0)
        return 0

    lax.fori_loop(1, NS, accum, 0)

    def comp(i, _):
        d = dslice[pl.ds(i * L, L)]
        dm = jnp.maximum(d, jnp.float32(1.0))
        y = _fast_rsqrt(dm)
        ybuf[pl.ds(i * L, L)] = y
        sqbuf[pl.ds(i * L, L)] = dm * y
        tbuf[pl.ds(i * L, L)] = y * y
        return 0

    lax.fori_loop(0, RPT // L, comp, 0)

    def finish(tab, z0, dsq, sq):
        pltpu.sync_copy(tbuf, dsq.at[pl.ds(s * RPT, RPT)])
        pltpu.sync_copy(sqbuf, sq.at[pl.ds(s * RPT, RPT)])

        def ch(cb, _):
            r0 = s * RPT + cb * RCHUNK
            pltpu.sync_copy(tab.at[pl.ds(r0, RCHUNK)], rowb)

            def rr(r, _):
                ii = jnp.full((L,), cb * RCHUNK + r, _i32)
                stage[r, :] = rowb[r, :] * plsc.load_gather(ybuf, [ii])
                return 0

            lax.fori_loop(0, RCHUNK, rr, 0)
            pltpu.sync_copy(stage, z0.at[pl.ds(r0, RCHUNK)])
            return 0

        lax.fori_loop(0, NCHUNKS, ch, 0)

    @pl.when(c == 0)
    def _():
        finish(ut, z0u, dsqu, squ)

    @pl.when(c == 1)
    def _():
        finish(it, z0i, dsqi, sqi)


_deg_z0 = pl.kernel(
    _deg_z0_body,
    out_type=[
        jax.ShapeDtypeStruct((NPAD, EMB), _f32),
        jax.ShapeDtypeStruct((NPAD, EMB), _f32),
        jax.ShapeDtypeStruct((NPAD,), _f32),
        jax.ShapeDtypeStruct((NPAD,), _f32),
        jax.ShapeDtypeStruct((NPAD,), _f32),
        jax.ShapeDtypeStruct((NPAD,), _f32),
    ],
    mesh=_mesh,
    compiler_params=pltpu.CompilerParams(
        needs_layout_passes=False, use_tc_tiling_on_sc=False
    ),
    scratch_types=[
        pltpu.VMEM_SHARED((NS, NPAD), _f32),
        pltpu.VMEM((NPAD,), _f32),
        pltpu.VMEM((1024,), _i32),
        pltpu.VMEM((RCHUNK, EMB), _f32),
        pltpu.VMEM((RCHUNK, EMB), _f32),
        pltpu.VMEM((RPT,), _f32),
        pltpu.VMEM((RPT,), _f32),
        pltpu.VMEM((RPT,), _f32),
        pltpu.VMEM((RPT,), _f32),
        pltpu.VMEM((RPT,), _f32),
    ],
)


def _side(final, s, dst2, src2, zsrc, dsq, acc, sq, oz, oa,
          s_sh, dstb0, srcb0, rows0, dstb1, srcb1, rows1, sbuf, abuf,
          ostage, astage, dsqb, sqb, semi, semg, sems):
    z16 = jnp.zeros((L,), _f32)

    def zz(i, _):
        astage[i, :] = z16
        return 0

    lax.fori_loop(0, RCHUNK, zz, 0)

    def zs(cb, _):
        pltpu.sync_copy(astage, s_sh.at[pl.ds(s * RPT + cb * RCHUNK, RCHUNK)])
        return 0

    lax.fori_loop(0, NCHUNKS, zs, 0)
    plsc.subcore_barrier()

    ept = (dst2.shape[0] - 1024) // NS
    base = s * ept
    n = ept // 1024

    def idx_copy(g, db, sb):
        pltpu.async_copy(dst2.at[pl.ds(base + g * 1024, 1024)], db, semi)
        pltpu.async_copy(src2.at[pl.ds(base + g * 1024, 1024)], sb, semi)

    def idx_wait(g, db, sb):
        pltpu.make_async_copy(
            dst2.at[pl.ds(base + g * 1024, 1024)], db, semi).wait()
        pltpu.make_async_copy(
            src2.at[pl.ds(base + g * 1024, 1024)], sb, semi).wait()

    def fire_gathers(sb, rw):
        pltpu.async_copy(zsrc.at[sb], rw, semg)

    def wait_gathers(sb, rw):
        pltpu.make_async_copy(zsrc.at[sb], rw, semg).wait()

    def fire_scatters(db, rw):
        pltpu.async_copy(rw, s_sh.at[db], sems, add=True)

    def wait_scatters(db, rw):
        pltpu.make_async_copy(rw, s_sh.at[db], sems).wait()

    bufs = ((dstb0, srcb0, rows0), (dstb1, srcb1, rows1))
    idx_copy(0, dstb0, srcb0)

    def piter(gg, _):
        for h in (0, 1):
            g = 2 * gg + h
            db, sb, rw = bufs[h]
            odb, osb, orw = bufs[1 - h]
            idx_wait(g, db, sb)
            fire_gathers(sb, rw)
            if h == 0:
                @pl.when(gg > 0)
                def _():
                    wait_scatters(odb, orw)
            else:
                wait_scatters(odb, orw)
            idx_copy(g + 1, odb, osb)
            wait_gathers(sb, rw)
            fire_scatters(db, rw)
        return 0

    lax.fori_loop(0, n // 2, piter, 0)
    wait_scatters(dstb1, rows1)
    idx_wait(n, dstb0, srcb0)
    plsc.subcore_barrier()

    pltpu.sync_copy(dsq.at[pl.ds(s * RPT, RPT)], dsqb)
    if final:
        pltpu.sync_copy(sq.at[pl.ds(s * RPT, RPT)], sqb)
    third = jnp.float32(1.0 / 3.0)

    def ch(cb, _):
        r0 = s * RPT + cb * RCHUNK
        pltpu.sync_copy(s_sh.at[pl.ds(r0, RCHUNK)], sbuf)
        pltpu.sync_copy(acc.at[pl.ds(r0, RCHUNK)], abuf)

        def rr(r, _):
            ii = jnp.full((L,), cb * RCHUNK + r, _i32)
            zrow = sbuf[r, :] * plsc.load_gather(dsqb, [ii])
            arow = abuf[r, :] + zrow
            if final:
                ostage[r, :] = arow * plsc.load_gather(sqb, [ii]) * third
            else:
                ostage[r, :] = zrow
                astage[r, :] = arow
            return 0

        lax.fori_loop(0, RCHUNK, rr, 0)
        pltpu.sync_copy(ostage, oz.at[pl.ds(r0, RCHUNK)])
        if not final:
            pltpu.sync_copy(astage, oa.at[pl.ds(r0, RCHUNK)])
        return 0

    lax.fori_loop(0, NCHUNKS, ch, 0)


def _layer_body(zu, zi, u2, v2, dsqu, dsqi, accu, acci,
                nzu, nzi, nau, nai, *scratch):
    c = lax.axis_index("c")
    s = lax.axis_index("s")

    @pl.when(c == 0)
    def _():
        _side(False, s, u2, v2, zi, dsqu, accu, None, nzu, nau, *scratch)

    @pl.when(c == 1)
    def _():
        _side(False, s, v2, u2, zu, dsqi, acci, None, nzi, nai, *scratch)


def _final_body(zu, zi, u2, v2, dsqu, dsqi, accu, acci, squ, sqi,
                eu, ei, *scratch):
    c = lax.axis_index("c")
    s = lax.axis_index("s")

    @pl.when(c == 0)
    def _():
        _side(True, s, u2, v2, zi, dsqu, accu, squ, eu, None, *scratch)

    @pl.when(c == 1)
    def _():
        _side(True, s, v2, u2, zu, dsqi, acci, sqi, ei, None, *scratch)


_layer_scratch = [
    pltpu.VMEM_SHARED((NPAD, EMB), _f32),
    pltpu.VMEM((1024,), _i32),
    pltpu.VMEM((1024,), _i32),
    pltpu.VMEM((1024, EMB), _f32),
    pltpu.VMEM((1024,), _i32),
    pltpu.VMEM((1024,), _i32),
    pltpu.VMEM((1024, EMB), _f32),
    pltpu.VMEM((RCHUNK, EMB), _f32),
    pltpu.VMEM((RCHUNK, EMB), _f32),
    pltpu.VMEM((RCHUNK, EMB), _f32),
    pltpu.VMEM((RCHUNK, EMB), _f32),
    pltpu.VMEM((RPT,), _f32),
    pltpu.VMEM((RPT,), _f32),
    pltpu.SemaphoreType.DMA,
    pltpu.SemaphoreType.DMA,
    pltpu.SemaphoreType.DMA,
]

_layer = pl.kernel(
    _layer_body,
    out_type=[jax.ShapeDtypeStruct((NPAD, EMB), _f32)] * 4,
    mesh=_mesh,
    compiler_params=pltpu.CompilerParams(
        needs_layout_passes=False, use_tc_tiling_on_sc=False
    ),
    scratch_types=_layer_scratch,
)

_final = pl.kernel(
    _final_body,
    out_type=[jax.ShapeDtypeStruct((NPAD, EMB), _f32)] * 2,
    mesh=_mesh,
    compiler_params=pltpu.CompilerParams(
        needs_layout_passes=False, use_tc_tiling_on_sc=False
    ),
    scratch_types=_layer_scratch,
)


def kernel(user_table, item_table, edge_index):
    E = edge_index.shape[1]
    n_chunks = -(-E // (NS * 1024))
    n_chunks += n_chunks % 2
    Epad = n_chunks * NS * 1024 + 1024
    u = edge_index[0].astype(_i32)
    v = edge_index[1].astype(_i32)
    pad = jnp.full((Epad - E,), NU, _i32)
    u2 = jnp.concatenate([u, pad])
    v2 = jnp.concatenate([v, pad])
    ut = jnp.pad(user_table, ((0, NPAD - NU), (0, 0)))
    it = jnp.pad(item_table, ((0, NPAD - NI), (0, 0)))

    z0u, z0i, dsqu, dsqi, squ, sqi = _deg_z0(ut, it, u2, v2)
    z1u, z1i, a1u, a1i = _layer(z0u, z0i, u2, v2, dsqu, dsqi, z0u, z0i)
    eu, ei = _final(z1u, z1i, u2, v2, dsqu, dsqi, a1u, a1i, squ, sqi)
    return eu[:NU], ei[:NI]


# speedup vs baseline: 152.2062x; 1.0687x over previous
import functools

import jax
import jax.numpy as jnp
from jax import lax
from jax.experimental import pallas as pl
from jax.experimental.pallas import tpu as pltpu
from jax.experimental.pallas import tpu_sc as plsc

NU = 50000
NI = 50000
EMB = 16
NC = 2
NS = 16
L = 16

RPT = 3200
NPAD = NS * RPT
RCHUNK = 320
NCHUNKS = RPT // RCHUNK

_f32 = jnp.float32
_i32 = jnp.int32

_mesh = plsc.VectorSubcoreMesh(
    core_axis_name="c", subcore_axis_name="s", num_cores=NC, num_subcores=NS
)


def _zero_vec(ref, nwords):
    z16 = jnp.zeros((L,), _f32)

    def zz(i, _):
        ref[pl.ds(i * L, L)] = z16
        return 0

    lax.fori_loop(0, nwords // L, zz, 0)


def _fast_rsqrt(x):
    xi = plsc.bitcast(x, _i32)
    xi = jnp.int32(0x5F3759DF) - (xi >> 1)
    y = plsc.bitcast(xi, _f32)
    hx = x * jnp.float32(0.5)
    y = y * (jnp.float32(1.5) - hx * y * y)
    y = y * (jnp.float32(1.5) - hx * y * y)
    y = y * (jnp.float32(1.5) - hx * y * y)
    return y


def _deg_z0_body(ut, it, u2, v2, z0u, z0i, dsqu, dsqi, squ, sqi,
                 deg_parts, hist, idxb0, idxb1, rowb, stage, dslice,
                 pbuf0, pbuf1, sqbuf, ybuf, tbuf, sem0, sem1):
    c = lax.axis_index("c")
    s = lax.axis_index("s")
    one16 = jnp.ones((L,), _f32)

    _zero_vec(hist, NPAD)

    def histogram(dst2):
        ept = (dst2.shape[0] - 1024) // NS
        base = s * ept
        n = ept // 1024
        bufs = ((idxb0, sem0), (idxb1, sem1))

        pltpu.async_copy(dst2.at[pl.ds(base, 1024)], idxb0, sem0)

        def chunk(gg, _):
            for h in (0, 1):
                g = 2 * gg + h
                ib, sm = bufs[h]
                ob, osm = bufs[1 - h]
                pltpu.make_async_copy(
                    dst2.at[pl.ds(base + g * 1024, 1024)], ib, sm).wait()
                pltpu.async_copy(
                    dst2.at[pl.ds(base + (g + 1) * 1024, 1024)], ob, osm)

                def kloop(kk, _):
                    idx = ib[pl.ds(kk * L, L)]
                    plsc.addupdate_scatter(hist, [idx], one16)
                    return 0

                lax.fori_loop(0, 1024 // L, kloop, 0)
            return 0

        lax.fori_loop(0, n // 2, chunk, 0)
        pltpu.make_async_copy(
            dst2.at[pl.ds(base + n * 1024, 1024)], idxb0, sem0).wait()

    @pl.when(c == 0)
    def _():
        histogram(u2)

    @pl.when(c == 1)
    def _():
        histogram(v2)

    pltpu.sync_copy(hist, deg_parts.at[c * NS + s])
    plsc.subcore_barrier()

    pltpu.sync_copy(deg_parts.at[c * NS, pl.ds(s * RPT, RPT)], dslice)
    pbufs = ((pbuf0, sem0), (pbuf1, sem1))
    pltpu.async_copy(deg_parts.at[c * NS + 1, pl.ds(s * RPT, RPT)], pbuf0, sem0)

    def accum(tt, _):
        for h in (0, 1):
            t = 1 + 2 * tt + h
            pb, sm = pbufs[h]
            ob, osm = pbufs[1 - h]
            pltpu.make_async_copy(
                deg_parts.at[c * NS + t, pl.ds(s * RPT, RPT)], pb, sm).wait()

            @pl.when(t + 1 < NS)
            def _():
                pltpu.async_copy(
                    deg_parts.at[c * NS + t + 1, pl.ds(s * RPT, RPT)], ob, osm)

            def avec(i, _):
                dslice[pl.ds(i * L, L)] = (
                    dslice[pl.ds(i * L, L)] + pb[pl.ds(i * L, L)]
                )
                return 0

            lax.fori_loop(0, RPT // L, avec, 0)
        return 0

    lax.fori_loop(0, (NS - 1) // 2, accum, 0)
    pltpu.make_async_copy(
        deg_parts.at[c * NS + 15, pl.ds(s * RPT, RPT)], pbuf0, sem0).wait()

    def avec15(i, _):
        dslice[pl.ds(i * L, L)] = (
            dslice[pl.ds(i * L, L)] + pbuf0[pl.ds(i * L, L)]
        )
        return 0

    lax.fori_loop(0, RPT // L, avec15, 0)

    def comp(i, _):
        d = dslice[pl.ds(i * L, L)]
        dm = jnp.maximum(d, jnp.float32(1.0))
        y = _fast_rsqrt(dm)
        ybuf[pl.ds(i * L, L)] = y
        sqbuf[pl.ds(i * L, L)] = dm * y
        tbuf[pl.ds(i * L, L)] = y * y
        return 0

    lax.fori_loop(0, RPT // L, comp, 0)

    def finish(tab, z0, dsq, sq):
        pltpu.sync_copy(tbuf, dsq.at[pl.ds(s * RPT, RPT)])
        pltpu.sync_copy(sqbuf, sq.at[pl.ds(s * RPT, RPT)])

        def ch(cb, _):
            r0 = s * RPT + cb * RCHUNK
            pltpu.sync_copy(tab.at[pl.ds(r0, RCHUNK)], rowb)

            def rr(r, _):
                ii = jnp.full((L,), cb * RCHUNK + r, _i32)
                stage[r, :] = rowb[r, :] * plsc.load_gather(ybuf, [ii])
                return 0

            lax.fori_loop(0, RCHUNK, rr, 0)
            pltpu.sync_copy(stage, z0.at[pl.ds(r0, RCHUNK)])
            return 0

        lax.fori_loop(0, NCHUNKS, ch, 0)

    @pl.when(c == 0)
    def _():
        finish(ut, z0u, dsqu, squ)

    @pl.when(c == 1)
    def _():
        finish(it, z0i, dsqi, sqi)


_deg_z0 = pl.kernel(
    _deg_z0_body,
    out_type=[
        jax.ShapeDtypeStruct((NPAD, EMB), _f32),
        jax.ShapeDtypeStruct((NPAD, EMB), _f32),
        jax.ShapeDtypeStruct((NPAD,), _f32),
        jax.ShapeDtypeStruct((NPAD,), _f32),
        jax.ShapeDtypeStruct((NPAD,), _f32),
        jax.ShapeDtypeStruct((NPAD,), _f32),
        jax.ShapeDtypeStruct((NC * NS, NPAD), _f32),
    ],
    mesh=_mesh,
    compiler_params=pltpu.CompilerParams(
        needs_layout_passes=False, use_tc_tiling_on_sc=False
    ),
    scratch_types=[
        pltpu.VMEM((NPAD,), _f32),
        pltpu.VMEM((1024,), _i32),
        pltpu.VMEM((1024,), _i32),
        pltpu.VMEM((RCHUNK, EMB), _f32),
        pltpu.VMEM((RCHUNK, EMB), _f32),
        pltpu.VMEM((RPT,), _f32),
        pltpu.VMEM((RPT,), _f32),
        pltpu.VMEM((RPT,), _f32),
        pltpu.VMEM((RPT,), _f32),
        pltpu.VMEM((RPT,), _f32),
        pltpu.VMEM((RPT,), _f32),
        pltpu.SemaphoreType.DMA,
        pltpu.SemaphoreType.DMA,
    ],
)


def _side(final, s, dst2, src2, zsrc, dsq, acc, sq, oz, oa,
          s_sh, dstb0, srcb0, rows0, dstb1, srcb1, rows1, sbuf, abuf,
          ostage, astage, dsqb, sqb, semi, semg, sems):
    z16 = jnp.zeros((L,), _f32)

    def zz(i, _):
        astage[i, :] = z16
        return 0

    lax.fori_loop(0, RCHUNK, zz, 0)

    def zs(cb, _):
        pltpu.sync_copy(astage, s_sh.at[pl.ds(s * RPT + cb * RCHUNK, RCHUNK)])
        return 0

    lax.fori_loop(0, NCHUNKS, zs, 0)
    plsc.subcore_barrier()

    ept = (dst2.shape[0] - 1024) // NS
    base = s * ept
    n = ept // 1024

    def idx_copy(g, db, sb):
        pltpu.async_copy(dst2.at[pl.ds(base + g * 1024, 1024)], db, semi)
        pltpu.async_copy(src2.at[pl.ds(base + g * 1024, 1024)], sb, semi)

    def idx_wait(g, db, sb):
        pltpu.make_async_copy(
            dst2.at[pl.ds(base + g * 1024, 1024)], db, semi).wait()
        pltpu.make_async_copy(
            src2.at[pl.ds(base + g * 1024, 1024)], sb, semi).wait()

    def fire_gathers(sb, rw):
        pltpu.async_copy(zsrc.at[sb], rw, semg)

    def wait_gathers(sb, rw):
        pltpu.make_async_copy(zsrc.at[sb], rw, semg).wait()

    def fire_scatters(db, rw):
        pltpu.async_copy(rw, s_sh.at[db], sems, add=True)

    def wait_scatters(db, rw):
        pltpu.make_async_copy(rw, s_sh.at[db], sems).wait()

    bufs = ((dstb0, srcb0, rows0), (dstb1, srcb1, rows1))
    idx_copy(0, dstb0, srcb0)

    def piter(gg, _):
        for h in (0, 1):
            g = 2 * gg + h
            db, sb, rw = bufs[h]
            odb, osb, orw = bufs[1 - h]
            idx_wait(g, db, sb)
            fire_gathers(sb, rw)
            if h == 0:
                @pl.when(gg > 0)
                def _():
                    wait_scatters(odb, orw)
            else:
                wait_scatters(odb, orw)
            idx_copy(g + 1, odb, osb)
            wait_gathers(sb, rw)
            fire_scatters(db, rw)
        return 0

    lax.fori_loop(0, n // 2, piter, 0)
    wait_scatters(dstb1, rows1)
    idx_wait(n, dstb0, srcb0)
    plsc.subcore_barrier()

    pltpu.sync_copy(dsq.at[pl.ds(s * RPT, RPT)], dsqb)
    if final:
        pltpu.sync_copy(sq.at[pl.ds(s * RPT, RPT)], sqb)
    third = jnp.float32(1.0 / 3.0)

    def ch(cb, _):
        r0 = s * RPT + cb * RCHUNK
        pltpu.sync_copy(s_sh.at[pl.ds(r0, RCHUNK)], sbuf)
        pltpu.sync_copy(acc.at[pl.ds(r0, RCHUNK)], abuf)

        def rr(r, _):
            ii = jnp.full((L,), cb * RCHUNK + r, _i32)
            zrow = sbuf[r, :] * plsc.load_gather(dsqb, [ii])
            arow = abuf[r, :] + zrow
            if final:
                ostage[r, :] = arow * plsc.load_gather(sqb, [ii]) * third
            else:
                ostage[r, :] = zrow
                astage[r, :] = arow
            return 0

        lax.fori_loop(0, RCHUNK, rr, 0)
        pltpu.sync_copy(ostage, oz.at[pl.ds(r0, RCHUNK)])
        if not final:
            pltpu.sync_copy(astage, oa.at[pl.ds(r0, RCHUNK)])
        return 0

    lax.fori_loop(0, NCHUNKS, ch, 0)


def _layer_body(zu, zi, u2, v2, dsqu, dsqi, accu, acci,
                nzu, nzi, nau, nai, *scratch):
    c = lax.axis_index("c")
    s = lax.axis_index("s")

    @pl.when(c == 0)
    def _():
        _side(False, s, u2, v2, zi, dsqu, accu, None, nzu, nau, *scratch)

    @pl.when(c == 1)
    def _():
        _side(False, s, v2, u2, zu, dsqi, acci, None, nzi, nai, *scratch)


def _final_body(zu, zi, u2, v2, dsqu, dsqi, accu, acci, squ, sqi,
                eu, ei, *scratch):
    c = lax.axis_index("c")
    s = lax.axis_index("s")

    @pl.when(c == 0)
    def _():
        _side(True, s, u2, v2, zi, dsqu, accu, squ, eu, None, *scratch)

    @pl.when(c == 1)
    def _():
        _side(True, s, v2, u2, zu, dsqi, acci, sqi, ei, None, *scratch)


_layer_scratch = [
    pltpu.VMEM_SHARED((NPAD, EMB), _f32),
    pltpu.VMEM((1024,), _i32),
    pltpu.VMEM((1024,), _i32),
    pltpu.VMEM((1024, EMB), _f32),
    pltpu.VMEM((1024,), _i32),
    pltpu.VMEM((1024,), _i32),
    pltpu.VMEM((1024, EMB), _f32),
    pltpu.VMEM((RCHUNK, EMB), _f32),
    pltpu.VMEM((RCHUNK, EMB), _f32),
    pltpu.VMEM((RCHUNK, EMB), _f32),
    pltpu.VMEM((RCHUNK, EMB), _f32),
    pltpu.VMEM((RPT,), _f32),
    pltpu.VMEM((RPT,), _f32),
    pltpu.SemaphoreType.DMA,
    pltpu.SemaphoreType.DMA,
    pltpu.SemaphoreType.DMA,
]

_layer = pl.kernel(
    _layer_body,
    out_type=[jax.ShapeDtypeStruct((NPAD, EMB), _f32)] * 4,
    mesh=_mesh,
    compiler_params=pltpu.CompilerParams(
        needs_layout_passes=False, use_tc_tiling_on_sc=False
    ),
    scratch_types=_layer_scratch,
)

_final = pl.kernel(
    _final_body,
    out_type=[jax.ShapeDtypeStruct((NPAD, EMB), _f32)] * 2,
    mesh=_mesh,
    compiler_params=pltpu.CompilerParams(
        needs_layout_passes=False, use_tc_tiling_on_sc=False
    ),
    scratch_types=_layer_scratch,
)


def kernel(user_table, item_table, edge_index):
    E = edge_index.shape[1]
    n_chunks = -(-E // (NS * 1024))
    n_chunks += n_chunks % 2
    Epad = n_chunks * NS * 1024 + 1024
    u = edge_index[0].astype(_i32)
    v = edge_index[1].astype(_i32)
    pad = jnp.full((Epad - E,), NU, _i32)
    u2 = jnp.concatenate([u, pad])
    v2 = jnp.concatenate([v, pad])
    ut = jnp.pad(user_table, ((0, NPAD - NU), (0, 0)))
    it = jnp.pad(item_table, ((0, NPAD - NI), (0, 0)))

    z0u, z0i, dsqu, dsqi, squ, sqi, _unused = _deg_z0(ut, it, u2, v2)
    z1u, z1i, a1u, a1i = _layer(z0u, z0i, u2, v2, dsqu, dsqi, z0u, z0i)
    eu, ei = _final(z1u, z1i, u2, v2, dsqu, dsqi, a1u, a1i, squ, sqi)
    return eu[:NU], ei[:NI]
